# Initial kernel scaffold; baseline (speedup 1.0000x reference)
import jax
import jax.numpy as jnp
from jax.experimental import pallas as pl


def kernel(node_features, edge_index, W_self1, W_neigh1, b1, W_self2, W_neigh2, b2):
    raise NotImplementedError("write your pallas kernel here")


# speedup vs baseline: 3.3850x; 3.3850x over previous
import jax
import jax.numpy as jnp
from jax import lax
from jax.experimental import pallas as pl
from jax.experimental.pallas import tpu as pltpu
from jax.experimental.pallas import tpu_sc as plsc

N_NODES = 10000
N_EDGES = 320000
D_FEAT = 128
N_CLS = 40

NP = 10240
DROW = NP // 128
DUMMY = 10008

NC = 2
NS = 16
NW = NC * NS
CH = 128
CPT = 80
EP = NW * CPT * CH
IG = 8
NIG = CPT // IG
NBUF = 2

RB = 1024


def _tc1_body(x_ref, w_ref, b1_ref, p1_ref, xs1_ref):
    pw = jnp.dot(x_ref[...], w_ref[...], preferred_element_type=jnp.float32)
    p1_ref[...] = pw[:, :D_FEAT]
    xs1_ref[...] = pw[:, D_FEAT:] + b1_ref[...]


def _tc1(x_pad, wcat, b1row):
    return pl.pallas_call(
        _tc1_body,
        grid=(NP // RB,),
        in_specs=[
            pl.BlockSpec((RB, D_FEAT), lambda i: (i, 0)),
            pl.BlockSpec((D_FEAT, 2 * D_FEAT), lambda i: (0, 0)),
            pl.BlockSpec((1, D_FEAT), lambda i: (0, 0)),
        ],
        out_specs=[
            pl.BlockSpec((RB, D_FEAT), lambda i: (i, 0)),
            pl.BlockSpec((RB, D_FEAT), lambda i: (i, 0)),
        ],
        out_shape=[
            jax.ShapeDtypeStruct((NP, D_FEAT), jnp.float32),
            jax.ShapeDtypeStruct((NP, D_FEAT), jnp.float32),
        ],
    )(x_pad, wcat, b1row)


def _tc2_body(s1_ref, xs1_ref, deg_ref, ws2_ref, b2_ref, h_ref, hs2_ref):
    degc = jnp.maximum(deg_ref[...], 1.0)
    s1 = s1_ref[0] + s1_ref[1]
    h = jnp.tanh(xs1_ref[...] + s1 / degc)
    h_ref[...] = h
    hs2_ref[...] = jnp.dot(h, ws2_ref[...], preferred_element_type=jnp.float32) + b2_ref[...]


def _tc2(s1p, xs1, dega, ws2, b2row):
    return pl.pallas_call(
        _tc2_body,
        grid=(NP // RB,),
        in_specs=[
            pl.BlockSpec((2, RB, D_FEAT), lambda i: (0, i, 0)),
            pl.BlockSpec((RB, D_FEAT), lambda i: (i, 0)),
            pl.BlockSpec((RB, 1), lambda i: (i, 0)),
            pl.BlockSpec((D_FEAT, N_CLS), lambda i: (0, 0)),
            pl.BlockSpec((1, N_CLS), lambda i: (0, 0)),
        ],
        out_specs=[
            pl.BlockSpec((RB, D_FEAT), lambda i: (i, 0)),
            pl.BlockSpec((RB, N_CLS), lambda i: (i, 0)),
        ],
        out_shape=[
            jax.ShapeDtypeStruct((NP, D_FEAT), jnp.float32),
            jax.ShapeDtypeStruct((NP, N_CLS), jnp.float32),
        ],
    )(s1p, xs1, dega, ws2, b2row)


def _tc3_body(s2_ref, hs2_ref, deg_ref, wn2_ref, out_ref):
    degc = jnp.maximum(deg_ref[...], 1.0)
    agg2 = (s2_ref[0] + s2_ref[1]) / degc
    out_ref[...] = hs2_ref[...] + jnp.dot(
        agg2, wn2_ref[...], preferred_element_type=jnp.float32)


def _tc3(s2p, hs2, dega, wn2):
    return pl.pallas_call(
        _tc3_body,
        grid=(NP // RB,),
        in_specs=[
            pl.BlockSpec((2, RB, D_FEAT), lambda i: (0, i, 0)),
            pl.BlockSpec((RB, N_CLS), lambda i: (i, 0)),
            pl.BlockSpec((RB, 1), lambda i: (i, 0)),
            pl.BlockSpec((D_FEAT, N_CLS), lambda i: (0, 0)),
        ],
        out_specs=pl.BlockSpec((RB, N_CLS), lambda i: (i, 0)),
        out_shape=jax.ShapeDtypeStruct((NP, N_CLS), jnp.float32),
    )(s2p, hs2, dega, wn2)


def _make_sc_segsum(with_deg):
    mesh = plsc.VectorSubcoreMesh(core_axis_name="c", subcore_axis_name="s")
    rpt = NP // NS
    drpt = 16

    def body(table, srcs, dsts, zeros, *rest):
        if with_deg:
            (out, outdeg, sidx, didx, rows, isem, gsems, ssem,
             degv, idv, acc, degacc) = rest
        else:
            out, sidx, didx, rows, isem, gsems, ssem, acc = rest
        c = lax.axis_index("c")
        s = lax.axis_index("s")
        t = c * NS + s
        r0 = s * rpt
        zb = rows[0]
        pltpu.sync_copy(zeros, zb)
        for k in range(rpt // CH):
            pltpu.sync_copy(zb, acc.at[pl.ds(r0 + k * CH, CH)])
        if with_deg:
            pltpu.sync_copy(zeros.at[pl.ds(0, DROW)], degv)

            @pl.when(s < DROW // drpt)
            def _():
                pltpu.sync_copy(zb.at[pl.ds(0, drpt)],
                                degacc.at[pl.ds(s * drpt, drpt)])
            for g in range(DROW // 16):
                idv[pl.ds(g * 16, 16)] = lax.iota(jnp.int32, 16) + g * 16
        plsc.subcore_barrier()

        def igroup(ig, carry):
            j0 = ig * IG
            la = pltpu.async_copy(srcs.at[t, pl.ds(j0, IG)], sidx, isem)
            lb = pltpu.async_copy(dsts.at[t, pl.ds(j0, IG)], didx, isem)
            la.wait()
            lb.wait()
            gets = [
                pltpu.async_copy(table.at[sidx.at[b]], rows[b], gsems[b])
                for b in range(NBUF)
            ]
            if with_deg:
                def cnt(i, carry2):
                    idx16 = didx[i // 8, pl.ds((i % 8) * 16, 16)]
                    plsc.addupdate_scatter(
                        degv,
                        [lax.shift_right_logical(idx16, 7),
                         lax.bitwise_and(idx16, 127)],
                        jnp.full((16,), 1.0, jnp.float32))
                    return carry2
                lax.fori_loop(0, IG * 8, cnt, 0)

            def pair(g2, carry3):
                jj = g2 * NBUF
                puts = []
                for b in range(NBUF):
                    gets[b].wait()
                    puts.append(pltpu.async_copy(
                        rows[b], acc.at[didx.at[jj + b]], ssem, add=True))
                for b in range(NBUF):
                    puts[b].wait()

                @pl.when(g2 + 1 < IG // NBUF)
                def _():
                    for b in range(NBUF):
                        pltpu.async_copy(
                            table.at[sidx.at[jj + NBUF + b]], rows[b], gsems[b])
                return carry3

            lax.fori_loop(0, IG // NBUF, pair, 0)
            return carry

        lax.fori_loop(0, NIG, igroup, 0)
        if with_deg:
            pltpu.sync_copy(degv, degacc.at[idv], add=True)
        plsc.subcore_barrier()
        for k in range(rpt // CH):
            pltpu.sync_copy(acc.at[pl.ds(r0 + k * CH, CH)], rows[k % NBUF])
            pltpu.sync_copy(rows[k % NBUF], out.at[c, pl.ds(r0 + k * CH, CH)])
        if with_deg:
            @pl.when(s < DROW // drpt)
            def _():
                pltpu.sync_copy(degacc.at[pl.ds(s * drpt, drpt)],
                                rows[0].at[pl.ds(0, drpt)])
                pltpu.sync_copy(rows[0].at[pl.ds(0, drpt)],
                                outdeg.at[c, pl.ds(s * drpt, drpt)])

    out_type = [jax.ShapeDtypeStruct((NC, NP, D_FEAT), jnp.float32)]
    scratch = [
        pltpu.VMEM((IG, CH), jnp.int32),
        pltpu.VMEM((IG, CH), jnp.int32),
        [pltpu.VMEM((CH, D_FEAT), jnp.float32) for _ in range(NBUF)],
        pltpu.SemaphoreType.DMA,
        [pltpu.SemaphoreType.DMA for _ in range(NBUF)],
        pltpu.SemaphoreType.DMA,
    ]
    if with_deg:
        out_type.append(jax.ShapeDtypeStruct((NC, DROW, 128), jnp.float32))
        scratch += [
            pltpu.VMEM((DROW, 128), jnp.float32),
            pltpu.VMEM((DROW,), jnp.int32),
        ]
    scratch.append(pltpu.VMEM_SHARED((NP, D_FEAT), jnp.float32))
    if with_deg:
        scratch.append(pltpu.VMEM_SHARED((DROW, 128), jnp.float32))

    return pl.kernel(
        body,
        out_type=out_type if with_deg else out_type[0],
        mesh=mesh,
        scratch_types=scratch,
        compiler_params=pltpu.CompilerParams(needs_layout_passes=False),
    )


_sc_segsum_deg = _make_sc_segsum(True)
_sc_segsum = _make_sc_segsum(False)


def kernel(node_features, edge_index, W_self1, W_neigh1, b1, W_self2, W_neigh2, b2):
    src = edge_index[0].astype(jnp.int32)
    dst = edge_index[1].astype(jnp.int32)
    pad = EP - N_EDGES
    src3 = jnp.concatenate([src, jnp.full((pad,), DUMMY, jnp.int32)]).reshape(NW, CPT, CH)
    dst3 = jnp.concatenate([dst, jnp.full((pad,), DUMMY, jnp.int32)]).reshape(NW, CPT, CH)

    x_pad = jnp.pad(node_features, ((0, NP - N_NODES), (0, 0)))
    wcat = jnp.concatenate([W_neigh1, W_self1], axis=1)
    b1row = b1.reshape(1, D_FEAT)
    b2row = b2.reshape(1, N_CLS)
    zeros = jnp.zeros((CH, D_FEAT), jnp.float32)

    p1, xs1 = _tc1(x_pad, wcat, b1row)
    s1p, degp = _sc_segsum_deg(p1, src3, dst3, zeros)
    dega = (degp[0] + degp[1]).reshape(NP, 1)
    h, hs2 = _tc2(s1p, xs1, dega, W_self2, b2row)
    s2p = _sc_segsum(h, src3, dst3, zeros)
    logits = _tc3(s2p, hs2, dega, W_neigh2)
    return logits[:N_NODES]


# speedup vs baseline: 9.9672x; 2.9445x over previous
import jax
import jax.numpy as jnp
from jax import lax
from jax.experimental import pallas as pl
from jax.experimental.pallas import tpu as pltpu
from jax.experimental.pallas import tpu_sc as plsc

N_NODES = 10000
N_EDGES = 320000
D_FEAT = 128
N_CLS = 40

NP = 10240
DROW = NP // 128
DUMMY = 10008

NC = 2
NS = 16
NW = NC * NS
CH = 128
CPT = 80
EP = NW * CPT * CH
IG = 8
NIG = CPT // IG
NBUF = 2

RB = 1024


def _tc1_body(x_ref, w_ref, b1_ref, p1_ref, xs1_ref):
    pw = jnp.dot(x_ref[...], w_ref[...], preferred_element_type=jnp.float32)
    p1_ref[...] = pw[:, :D_FEAT]
    xs1_ref[...] = pw[:, D_FEAT:] + b1_ref[...]


def _tc1(x_pad, wcat, b1row):
    return pl.pallas_call(
        _tc1_body,
        grid=(NP // RB,),
        in_specs=[
            pl.BlockSpec((RB, D_FEAT), lambda i: (i, 0)),
            pl.BlockSpec((D_FEAT, 2 * D_FEAT), lambda i: (0, 0)),
            pl.BlockSpec((1, D_FEAT), lambda i: (0, 0)),
        ],
        out_specs=[
            pl.BlockSpec((RB, D_FEAT), lambda i: (i, 0)),
            pl.BlockSpec((RB, D_FEAT), lambda i: (i, 0)),
        ],
        out_shape=[
            jax.ShapeDtypeStruct((NP, D_FEAT), jnp.float32),
            jax.ShapeDtypeStruct((NP, D_FEAT), jnp.float32),
        ],
    )(x_pad, wcat, b1row)


def _tc2_body(s1_ref, xs1_ref, deg_ref, ws2_ref, b2_ref, h_ref, hs2_ref):
    degc = jnp.maximum(deg_ref[...], 1.0)
    s1 = s1_ref[0] + s1_ref[1]
    h = jnp.tanh(xs1_ref[...] + s1 / degc)
    h_ref[...] = h
    hs2_ref[...] = jnp.dot(h, ws2_ref[...], preferred_element_type=jnp.float32) + b2_ref[...]


def _tc2(s1p, xs1, dega, ws2, b2row):
    return pl.pallas_call(
        _tc2_body,
        grid=(NP // RB,),
        in_specs=[
            pl.BlockSpec((2, RB, D_FEAT), lambda i: (0, i, 0)),
            pl.BlockSpec((RB, D_FEAT), lambda i: (i, 0)),
            pl.BlockSpec((RB, 1), lambda i: (i, 0)),
            pl.BlockSpec((D_FEAT, N_CLS), lambda i: (0, 0)),
            pl.BlockSpec((1, N_CLS), lambda i: (0, 0)),
        ],
        out_specs=[
            pl.BlockSpec((RB, D_FEAT), lambda i: (i, 0)),
            pl.BlockSpec((RB, N_CLS), lambda i: (i, 0)),
        ],
        out_shape=[
            jax.ShapeDtypeStruct((NP, D_FEAT), jnp.float32),
            jax.ShapeDtypeStruct((NP, N_CLS), jnp.float32),
        ],
    )(s1p, xs1, dega, ws2, b2row)


def _tc3_body(s2_ref, hs2_ref, deg_ref, wn2_ref, out_ref):
    degc = jnp.maximum(deg_ref[...], 1.0)
    agg2 = (s2_ref[0] + s2_ref[1]) / degc
    out_ref[...] = hs2_ref[...] + jnp.dot(
        agg2, wn2_ref[...], preferred_element_type=jnp.float32)


def _tc3(s2p, hs2, dega, wn2):
    return pl.pallas_call(
        _tc3_body,
        grid=(NP // RB,),
        in_specs=[
            pl.BlockSpec((2, RB, D_FEAT), lambda i: (0, i, 0)),
            pl.BlockSpec((RB, N_CLS), lambda i: (i, 0)),
            pl.BlockSpec((RB, 1), lambda i: (i, 0)),
            pl.BlockSpec((D_FEAT, N_CLS), lambda i: (0, 0)),
        ],
        out_specs=pl.BlockSpec((RB, N_CLS), lambda i: (i, 0)),
        out_shape=jax.ShapeDtypeStruct((NP, N_CLS), jnp.float32),
    )(s2p, hs2, dega, wn2)


def _make_sc_segsum(with_deg):
    mesh = plsc.VectorSubcoreMesh(core_axis_name="c", subcore_axis_name="s")
    rpt = NP // NS
    drpt = 16

    def body(table, srcs, dsts, zeros, *rest):
        if with_deg:
            (out, outdeg, sidx, didx, rows, isems, gsems, ssems,
             degv, idv, acc, degacc) = rest
        else:
            out, sidx, didx, rows, isems, gsems, ssems, acc = rest
        c = lax.axis_index("c")
        s = lax.axis_index("s")
        t = c * NS + s
        r0 = s * rpt
        zb = rows[0]
        pltpu.sync_copy(zeros, zb)
        for k in range(rpt // CH):
            pltpu.sync_copy(zb, acc.at[pl.ds(r0 + k * CH, CH)])
        if with_deg:
            pltpu.sync_copy(zeros.at[pl.ds(0, DROW)], degv)

            @pl.when(s < DROW // drpt)
            def _():
                pltpu.sync_copy(zb.at[pl.ds(0, drpt)],
                                degacc.at[pl.ds(s * drpt, drpt)])
            for g in range(DROW // 16):
                idv[pl.ds(g * 16, 16)] = lax.iota(jnp.int32, 16) + g * 16
        plsc.subcore_barrier()

        pltpu.async_copy(srcs.at[t, pl.ds(0, IG)], sidx[0], isems[0])
        pltpu.async_copy(dsts.at[t, pl.ds(0, IG)], didx[0], isems[0])

        def sgroup(sg, carry):
            for par in range(2):
                g = sg * 2 + par
                j0 = g * IG
                si = sidx[par]
                di = didx[par]
                pltpu.make_async_copy(srcs.at[t, pl.ds(j0, IG)], si,
                                      isems[par]).wait()
                pltpu.make_async_copy(dsts.at[t, pl.ds(j0, IG)], di,
                                      isems[par]).wait()

                @pl.when(g + 1 < NIG)
                def _():
                    jn = (g + 1) * IG
                    pltpu.async_copy(srcs.at[t, pl.ds(jn, IG)],
                                     sidx[1 - par], isems[1 - par])
                    pltpu.async_copy(dsts.at[t, pl.ds(jn, IG)],
                                     didx[1 - par], isems[1 - par])

                gets = [
                    pltpu.async_copy(table.at[si.at[b]], rows[b], gsems[b])
                    for b in range(NBUF)
                ]
                if with_deg:
                    def cnt(i, carry2):
                        idx16 = di[i // 8, pl.ds((i % 8) * 16, 16)]
                        plsc.addupdate_scatter(
                            degv,
                            [lax.shift_right_logical(idx16, 7),
                             lax.bitwise_and(idx16, 127)],
                            jnp.full((16,), 1.0, jnp.float32))
                        return carry2
                    lax.fori_loop(0, IG * 8, cnt, 0)

                def pair(g2, carry3):
                    jj = g2 * NBUF
                    puts = []
                    for b in range(NBUF):
                        gets[b].wait()
                        puts.append(pltpu.async_copy(
                            rows[b], acc.at[di.at[jj + b]], ssems[b], add=True))

                    @pl.when(g2 + 1 < IG // NBUF)
                    def _():
                        for b in range(NBUF):
                            puts[b].wait()
                            pltpu.async_copy(
                                table.at[si.at[jj + NBUF + b]], rows[b],
                                gsems[b])

                    @pl.when(g2 + 1 >= IG // NBUF)
                    def _():
                        for b in range(NBUF):
                            puts[b].wait()
                    return carry3

                lax.fori_loop(0, IG // NBUF, pair, 0)
            return carry

        lax.fori_loop(0, NIG // 2, sgroup, 0)
        if with_deg:
            pltpu.sync_copy(degv, degacc.at[idv], add=True)
        plsc.subcore_barrier()
        for k in range(rpt // CH):
            pltpu.sync_copy(acc.at[pl.ds(r0 + k * CH, CH)], rows[k % NBUF])
            pltpu.sync_copy(rows[k % NBUF], out.at[c, pl.ds(r0 + k * CH, CH)])
        if with_deg:
            @pl.when(s < DROW // drpt)
            def _():
                pltpu.sync_copy(degacc.at[pl.ds(s * drpt, drpt)],
                                rows[0].at[pl.ds(0, drpt)])
                pltpu.sync_copy(rows[0].at[pl.ds(0, drpt)],
                                outdeg.at[c, pl.ds(s * drpt, drpt)])

    out_type = [jax.ShapeDtypeStruct((NC, NP, D_FEAT), jnp.float32)]
    scratch = [
        [pltpu.VMEM((IG, CH), jnp.int32) for _ in range(2)],
        [pltpu.VMEM((IG, CH), jnp.int32) for _ in range(2)],
        [pltpu.VMEM((CH, D_FEAT), jnp.float32) for _ in range(NBUF)],
        [pltpu.SemaphoreType.DMA for _ in range(2)],
        [pltpu.SemaphoreType.DMA for _ in range(NBUF)],
        [pltpu.SemaphoreType.DMA for _ in range(NBUF)],
    ]
    if with_deg:
        out_type.append(jax.ShapeDtypeStruct((NC, DROW, 128), jnp.float32))
        scratch += [
            pltpu.VMEM((DROW, 128), jnp.float32),
            pltpu.VMEM((DROW,), jnp.int32),
        ]
    scratch.append(pltpu.VMEM_SHARED((NP, D_FEAT), jnp.float32))
    if with_deg:
        scratch.append(pltpu.VMEM_SHARED((DROW, 128), jnp.float32))

    return pl.kernel(
        body,
        out_type=out_type if with_deg else out_type[0],
        mesh=mesh,
        scratch_types=scratch,
        compiler_params=pltpu.CompilerParams(needs_layout_passes=False),
    )


_sc_segsum_deg = _make_sc_segsum(True)
_sc_segsum = _make_sc_segsum(False)


def kernel(node_features, edge_index, W_self1, W_neigh1, b1, W_self2, W_neigh2, b2):
    src = edge_index[0].astype(jnp.int32)
    dst = edge_index[1].astype(jnp.int32)
    pad = EP - N_EDGES
    ppt = pad // NW
    rpt_real = N_EDGES // NW
    dummy = (N_NODES + (jnp.arange(pad, dtype=jnp.int32) % (NP - N_NODES))
             ).reshape(NW, ppt)
    src3 = jnp.concatenate([src.reshape(NW, rpt_real), dummy], axis=1
                           ).reshape(NW, CPT, CH)
    dst3 = jnp.concatenate([dst.reshape(NW, rpt_real), dummy], axis=1
                           ).reshape(NW, CPT, CH)

    x_pad = jnp.pad(node_features, ((0, NP - N_NODES), (0, 0)))
    wcat = jnp.concatenate([W_neigh1, W_self1], axis=1)
    b1row = b1.reshape(1, D_FEAT)
    b2row = b2.reshape(1, N_CLS)
    zeros = jnp.zeros((CH, D_FEAT), jnp.float32)

    p1, xs1 = _tc1(x_pad, wcat, b1row)
    s1p, degp = _sc_segsum_deg(p1, src3, dst3, zeros)
    dega = (degp[0] + degp[1]).reshape(NP, 1)
    h, hs2 = _tc2(s1p, xs1, dega, W_self2, b2row)
    s2p = _sc_segsum(h, src3, dst3, zeros)
    logits = _tc3(s2p, hs2, dega, W_neigh2)
    return logits[:N_NODES]
